# baseline (speedup 1.0000x reference)
import functools

import jax
import jax.numpy as jnp
import numpy as np
from jax import lax
from jax.experimental import pallas as pl
from jax.experimental.pallas import tpu as pltpu
from jax.experimental.pallas import tpu_sc as plsc

N_EMBD = 768
HALF = N_EMBD // 2
VOCAB = 2048
B, S = 64, 1024
NA = 16384
NCT = 16384
NT = 4096
NROWS = NA + NT


BLK = 256
NBLK = NROWS // BLK

def _mlp_body(attr_ref, w1_ref, b1_ref, g_ref, be_ref, w2_ref, b2_ref, out_ref):
    a = attr_ref[...]
    it = lax.broadcasted_iota(jnp.int32, (1, HALF), 1).astype(jnp.float32)
    freqs = jnp.exp((-np.log(10000.0) / HALF) * it)
    m = a[:, 5:6]
    s = jnp.zeros((BLK, HALF), jnp.float32)
    c = jnp.zeros((BLK, HALF), jnp.float32)
    for k in range(5):
        ang = a[:, k:k + 1] * freqs
        sk, ck = jnp.sin(ang), jnp.cos(ang)
        if k >= 3:
            sk, ck = sk * m, ck * m
        s, c = s + sk, c + ck
    h = jnp.concatenate([s, c], axis=1)
    h1 = jnp.dot(h.astype(jnp.bfloat16), w1_ref[...],
                 preferred_element_type=jnp.float32) + b1_ref[...]
    mu = jnp.mean(h1, axis=1, keepdims=True)
    var = jnp.mean((h1 - mu) ** 2, axis=1, keepdims=True)
    h1 = (h1 - mu) * lax.rsqrt(var + 1e-5) * g_ref[...] + be_ref[...]
    h1 = jnp.maximum(h1, 0.0)
    out_ref[...] = jnp.dot(h1.astype(jnp.bfloat16), w2_ref[...],
                           preferred_element_type=jnp.float32) + b2_ref[...]


def _run_mlp(attrs, W1, b1, g, be, W2, b2):
    return pl.pallas_call(
        _mlp_body,
        grid=(NBLK,),
        in_specs=[
            pl.BlockSpec((BLK, 8), lambda i: (i, 0)),
            pl.BlockSpec((N_EMBD, 2 * N_EMBD), lambda i: (0, 0)),
            pl.BlockSpec((1, 2 * N_EMBD), lambda i: (0, 0)),
            pl.BlockSpec((1, 2 * N_EMBD), lambda i: (0, 0)),
            pl.BlockSpec((1, 2 * N_EMBD), lambda i: (0, 0)),
            pl.BlockSpec((2 * N_EMBD, N_EMBD), lambda i: (0, 0)),
            pl.BlockSpec((1, N_EMBD), lambda i: (0, 0)),
        ],
        out_specs=pl.BlockSpec((BLK, N_EMBD), lambda i: (i, 0)),
        out_shape=jax.ShapeDtypeStruct((NROWS, N_EMBD), jnp.float32),
    )(attrs, W1, b1, g, be, W2, b2)


NSTREAM = NCT + 5 * NA + NA + NA + NT + NT + NT
NSC, NSUB = 2, 16
NW = NSC * NSUB
SH = NSTREAM // NW
SEG = SH // 2
NKEY = 1024
KSH = 6
TROWS = (B * S) // NW
AROWS = TROWS // 32
BCAP = 40
PR = 48
BK = 32
SCAP = 36
DRAIN = 4096
ZROW = VOCAB + 71 + NROWS


def _rank_group(d, base0):
    k = d >> KSH
    cnt_run, lastm = plsc.scan_count(k)
    return k, cnt_run - base0, lastm


def _bin_body(idxs, dests, pool_i, pool_d, rst_out, segi, segd, bin_i, bin_d,
              cnt, rstart, cursor, semg):
    cid = lax.axis_index("c")
    sid = lax.axis_index("s")
    wid = sid * NSC + cid
    base = wid * SH
    lanes = lax.iota(jnp.int32, 16)
    zv16 = jnp.zeros((16,), jnp.int32)
    base0 = jnp.min(plsc.scan_count(zv16)[0])

    for b in range(NKEY // 16):
        cnt[pl.ds(b * 16, 16)] = zv16

    for s in range(SH // SEG):
        pltpu.sync_copy(dests.at[pl.ds(base + s * SEG, SEG)], segd)

        def grpA(g, _):
            d = segd[pl.ds(g * 16, 16)]
            k, rank, lastm = _rank_group(d, base0)
            plsc.addupdate_scatter(cnt, [k], rank + 1, mask=lastm)
            return 0

        lax.fori_loop(0, SEG // 16, grpA, 0)

    carry = jnp.int32(0)
    for b in range(NKEY // 16):
        v = cnt[pl.ds(b * 16, 16)]
        cs = plsc.cumsum(v)
        excl = cs - v + carry
        rstart[pl.ds(b * 16, 16)] = excl
        cursor[pl.ds(b * 16, 16)] = excl
        carry = carry + jnp.sum(v)
    plsc.store_scatter(rstart, [lanes + NKEY], jnp.full((16,), SH, jnp.int32))

    for s in range(SH // SEG):
        pltpu.sync_copy(dests.at[pl.ds(base + s * SEG, SEG)], segd)
        pltpu.sync_copy(idxs.at[pl.ds(base + s * SEG, SEG)], segi)

        def grpB(g, _):
            d = segd[pl.ds(g * 16, 16)]
            ix = segi[pl.ds(g * 16, 16)]
            k, rank, lastm = _rank_group(d, base0)
            pos = plsc.load_gather(cursor, [k]) + rank
            plsc.store_scatter(bin_i, [pos >> 7, pos & 127], ix)
            plsc.store_scatter(bin_d, [pos >> 7, pos & 127], d & (AROWS - 1))
            plsc.addupdate_scatter(cursor, [k], rank + 1, mask=lastm)
            return 0

        lax.fori_loop(0, SEG // 16, grpB, 0)

    pltpu.sync_copy(bin_i, pool_i.at[wid, pl.ds(0, BCAP)])
    pltpu.sync_copy(bin_d, pool_d.at[wid, pl.ds(0, BCAP)])
    pltpu.sync_copy(rstart, rst_out.at[pl.ds(wid * (NKEY + 16), NKEY + 16)])


def _run_bin(idx_all, dest_all):
    mesh = plsc.VectorSubcoreMesh(core_axis_name="c", subcore_axis_name="s")
    kfn = functools.partial(
        pl.kernel,
        compiler_params=pltpu.CompilerParams(needs_layout_passes=False),
        out_type=(
            jax.ShapeDtypeStruct((NW, PR, 128), jnp.int32),
            jax.ShapeDtypeStruct((NW, PR, 128), jnp.int32),
            jax.ShapeDtypeStruct((NW * (NKEY + 16),), jnp.int32),
        ),
        mesh=mesh,
        scratch_types=[
            pltpu.VMEM((SEG,), jnp.int32),
            pltpu.VMEM((SEG,), jnp.int32),
            pltpu.VMEM((BCAP, 128), jnp.int32),
            pltpu.VMEM((BCAP, 128), jnp.int32),
            pltpu.VMEM((NKEY,), jnp.int32),
            pltpu.VMEM((NKEY + 16,), jnp.int32),
            pltpu.VMEM((NKEY,), jnp.int32),
            pltpu.SemaphoreType.DMA,
        ],
    )(_bin_body)
    return kfn(idx_all, dest_all)


def _acc_body(table, pool_i, pool_d, rst, out, sts, sts_s, sl_i, sl_d,
              stg_i, stg_d, sidxA, rows, acc, semg):
    cid = lax.axis_index("c")
    sid = lax.axis_index("s")
    wid = sid * NSC + cid
    lanes = lax.iota(jnp.int32, 16)
    zvf = jnp.zeros((16,), jnp.float32)

    for s in range(NW):
        pltpu.sync_copy(rst.at[pl.ds(s * (NKEY + 16) + wid * 32, 48)],
                        sts.at[pl.ds(s * 48, 48)])
    for s in range(NW):
        for q in range(33):
            lq = lanes == (q & 15)
            half = sts[pl.ds(s * 48 + (q >> 4) * 16, 16)]
            sts_s[s * 33 + q] = jnp.sum(jnp.where(lq, half, 0))

    def _zero_acc():
        for r in range(AROWS + 1):
            def zwr(j, _, r=r):
                acc[r, pl.ds(j * 16, 16)] = zvf
                return 0
            lax.fori_loop(0, N_EMBD // 16, zwr, 0)

    def _drain(cnt):
        npad = (-cnt) & (BK - 1)
        for q in range(BK // 16):
            qpos = cnt + q * 16 + lanes
            qm = (q * 16 + lanes) < npad
            plsc.store_scatter(stg_i, [qpos >> 7, qpos & 127],
                               jnp.full((16,), ZROW, jnp.int32), mask=qm)
            plsc.store_scatter(stg_d, [qpos >> 7, qpos & 127],
                               jnp.full((16,), AROWS, jnp.int32), mask=qm)
        nbat = (cnt + npad) // BK

        def bl(j, _):
            s2 = j * BK
            r2 = s2 >> 7
            c2 = s2 & 127
            for h in range(BK // 16):
                plsc.store_scatter(sidxA, [lanes + h * 16],
                                   stg_i[r2, pl.ds(c2 + h * 16, 16)])
            pltpu.async_copy(table.at[sidxA], rows, semg).wait()
            for h in range(BK // 16):
                dv = stg_d[r2, pl.ds(c2 + h * 16, 16)]
                for q in range(16):
                    loc = jnp.sum(jnp.where(lanes == q, dv, 0))
                    rr = h * 16 + q

                    def addf(f, _, rr=rr, loc=loc):
                        plsc.addupdate(acc.at[loc, pl.ds(f * 64, 16)],
                                       rows[rr, pl.ds(f * 64, 16)])
                        plsc.addupdate(acc.at[loc, pl.ds(f * 64 + 16, 16)],
                                       rows[rr, pl.ds(f * 64 + 16, 16)])
                        plsc.addupdate(acc.at[loc, pl.ds(f * 64 + 32, 16)],
                                       rows[rr, pl.ds(f * 64 + 32, 16)])
                        plsc.addupdate(acc.at[loc, pl.ds(f * 64 + 48, 16)],
                                       rows[rr, pl.ds(f * 64 + 48, 16)])
                        return 0

                    lax.fori_loop(0, N_EMBD // 64, addf, 0)
            return 0

        lax.fori_loop(0, nbat, bl, 0)

    def do_sub(v, _):
        _zero_acc()

        def share_loop(s, off):
            w0 = sts_s[s * 33 + v]
            w1 = sts_s[s * 33 + v + 1]
            a0 = w0 >> 10
            nsl = jnp.where(w1 > w0, ((w1 - 1) >> 10) - a0 + 1, 0)

            def slab(q, off):
                sb = (a0 + q) * 1024
                pltpu.sync_copy(pool_i.at[s, pl.ds((a0 + q) * 8, 8)], sl_i)
                pltpu.sync_copy(pool_d.at[s, pl.ds((a0 + q) * 8, 8)], sl_d)
                g0 = lax.max(jnp.int32(0), (w0 - sb) >> 4)
                g1 = lax.min(jnp.int32(64), ((w1 - sb - 1) >> 4) + 1)

                def grp(h, off):
                    e = sb + h * 16 + lanes
                    m = (e >= w0) & (e < w1)
                    mi = m.astype(jnp.int32)
                    pos = off + plsc.cumsum(mi) - mi
                    plsc.store_scatter(stg_i, [pos >> 7, pos & 127],
                                       sl_i[h >> 3, pl.ds((h & 7) * 16, 16)],
                                       mask=m)
                    plsc.store_scatter(stg_d, [pos >> 7, pos & 127],
                                       sl_d[h >> 3, pl.ds((h & 7) * 16, 16)],
                                       mask=m)
                    return off + jnp.sum(mi)

                off = lax.fori_loop(g0, g1, grp, off)

                @pl.when(off >= DRAIN)
                def _():
                    _drain(off)

                return jnp.where(off >= DRAIN, 0, off)

            return lax.fori_loop(0, nsl, slab, off)

        cnt = lax.fori_loop(0, NW, share_loop, jnp.int32(0))

        @pl.when(cnt > 0)
        def _():
            _drain(cnt)

        pltpu.sync_copy(acc.at[pl.ds(0, AROWS)],
                        out.at[pl.ds(wid * TROWS + v * AROWS, AROWS)])
        return 0

    lax.fori_loop(0, 32, do_sub, 0)


def _run_acc(table, pool_i, pool_d, rst):
    mesh = plsc.VectorSubcoreMesh(core_axis_name="c", subcore_axis_name="s")
    kfn = functools.partial(
        pl.kernel,
        compiler_params=pltpu.CompilerParams(needs_layout_passes=False),
        out_type=jax.ShapeDtypeStruct((B * S, N_EMBD), jnp.float32),
        mesh=mesh,
        scratch_types=[
            pltpu.VMEM((NW * 48,), jnp.int32),
            pltpu.SMEM((NW * 33,), jnp.int32),
            pltpu.VMEM((8, 128), jnp.int32),
            pltpu.VMEM((8, 128), jnp.int32),
            pltpu.VMEM((SCAP, 128), jnp.int32),
            pltpu.VMEM((SCAP, 128), jnp.int32),
            pltpu.VMEM((BK,), jnp.int32),
            pltpu.VMEM((BK, N_EMBD), jnp.float32),
            pltpu.VMEM((AROWS + 1, N_EMBD), jnp.float32),
            pltpu.SemaphoreType.DMA,
        ],
    )(_acc_body)
    return kfn(table, pool_i, pool_d, rst)


def _run_scatter(table, idx_all, dest_all):
    pool_i, pool_d, rst = _run_bin(idx_all, dest_all)
    return _run_acc(table, pool_i, pool_d, rst)


def kernel(control_tokens, control_batch_idx, control_seq_idx, x_token,
           y_token, heading_token, width_token, length_token, agent_type_idx,
           agent_x, agent_y, agent_heading, agent_width, agent_length,
           agent_batch_idx, agent_seq_idx, tl_x, tl_y, tl_heading, tl_state,
           tl_index, tl_batch_idx, tl_seq_idx, token_embedding,
           agent_type_embedding, traffic_light_status_embedding,
           tl_index_embedding, W1, b1, ln_gamma, ln_beta, W2, b2):
    f32 = jnp.float32
    i32 = jnp.int32

    attrs = jnp.stack([
        jnp.concatenate([agent_x, tl_x]),
        jnp.concatenate([agent_y, tl_y]),
        jnp.concatenate([agent_heading, tl_heading]),
        jnp.concatenate([agent_width, jnp.zeros((NT,), f32)]),
        jnp.concatenate([agent_length, jnp.zeros((NT,), f32)]),
        jnp.concatenate([jnp.ones((NA,), f32), jnp.zeros((NT,), f32)]),
        jnp.zeros((NROWS,), f32),
        jnp.zeros((NROWS,), f32),
    ], axis=1)
    mlp_out = _run_mlp(attrs, W1.astype(jnp.bfloat16), b1[None, :],
                       ln_gamma[None, :], ln_beta[None, :],
                       W2.astype(jnp.bfloat16), b2[None, :])

    table = jnp.concatenate([
        token_embedding,
        agent_type_embedding,
        traffic_light_status_embedding,
        tl_index_embedding,
        mlp_out,
        jnp.zeros((1, N_EMBD), f32),
    ], axis=0)
    O_TYPE, O_ST, O_TLI, O_MLP = VOCAB, VOCAB + 3, VOCAB + 7, VOCAB + 71

    ctrl_d = control_batch_idx.astype(i32) * S + control_seq_idx.astype(i32)
    sidx = jnp.clip(agent_seq_idx.astype(i32), 1, S - 1)
    agent_d = agent_batch_idx.astype(i32) * S + sidx
    tl_d = tl_batch_idx.astype(i32) * S + tl_seq_idx.astype(i32)

    def ctok(t):
        return jnp.clip(t.astype(i32), 0, VOCAB - 1)

    idx_all = jnp.concatenate([
        ctok(control_tokens),
        ctok(x_token), ctok(y_token), ctok(heading_token),
        ctok(width_token), ctok(length_token),
        O_TYPE + jnp.clip(agent_type_idx.astype(i32), 0, 2),
        O_MLP + jnp.arange(NA, dtype=i32),
        O_ST + jnp.clip(tl_state.astype(i32), 0, 3),
        O_TLI + jnp.clip(tl_index.astype(i32), 0, 63),
        O_MLP + NA + jnp.arange(NT, dtype=i32),
    ])
    dest_all = jnp.concatenate([
        ctrl_d,
        agent_d, agent_d, agent_d, agent_d, agent_d,
        agent_d - 1,
        agent_d,
        tl_d, tl_d, tl_d,
    ])

    emb = _run_scatter(table, idx_all, dest_all)
    return emb.reshape(B, S, N_EMBD)
